# baseline (speedup 1.0000x reference)
import functools

import jax
import jax.numpy as jnp
from jax import lax
from jax.experimental import pallas as pl
from jax.experimental.pallas import tpu as pltpu
from jax.experimental.pallas import tpu_sc as plsc

_N = 2048
_LANES = 16
_NC = 2
_NS = 16
_NW = _NC * _NS
_SC_ROWS = 384
_ROWS_PER_W = _SC_ROWS // _NW
_MCHUNKS = _N // _LANES
_HT2 = (0.125, 0.5, 2.0, 8.0)

_TC_BLK = 128


def _sc_body(pred_hbm, true_hbm, mask_hbm, out_hbm,
             rawp, rawt, predc, truec, maskc, stage):
    c = lax.axis_index("c")
    s = lax.axis_index("s")
    wid = s * _NC + c
    row0 = wid * _ROWS_PER_W

    pltpu.sync_copy(pred_hbm, rawp)
    pltpu.sync_copy(true_hbm, rawt)
    win0 = pl.multiple_of((row0 // 8) * 8, 8)
    moff = row0 - win0
    pltpu.sync_copy(mask_hbm.at[pl.ds(win0, 16), :], maskc)

    iota3 = lax.iota(jnp.int32, _LANES) * 3

    def transpose_body(g, carry):
        base = g * (3 * _LANES)
        for d in range(3):
            truec[pl.ds(d * _N + g * _LANES, _LANES)] = plsc.load_gather(
                rawt, [iota3 + (base + d)]
            )
        for smp in range(5):
            for d in range(3):
                predc[pl.ds((smp * 3 + d) * _N + g * _LANES, _LANES)] = (
                    plsc.load_gather(
                        rawp, [iota3 + (smp * 3 * _N + base + d)]
                    )
                )
        return carry

    lax.fori_loop(0, _MCHUNKS, transpose_body, 0)

    zero16 = jnp.zeros((_LANES,), jnp.int32)

    def row_body(r, carry1):
        l3 = (row0 + r) * 3
        lf = jnp.full((_LANES,), l3, jnp.int32)
        txl = plsc.load_gather(rawt, [lf])
        tyl = plsc.load_gather(rawt, [lf + 1])
        tzl = plsc.load_gather(rawt, [lf + 2])
        psplat = []
        for smp in range(5):
            for d in range(3):
                psplat.append(
                    plsc.load_gather(rawp, [lf + (smp * 3 * _N + d)])
                )

        def m_body(mc, carry2):
            accs, npv = carry2
            o = mc * _LANES
            mv = maskc[moff + r, pl.ds(o, _LANES)]
            txm = truec[pl.ds(o, _LANES)]
            tym = truec[pl.ds(_N + o, _LANES)]
            tzm = truec[pl.ds(2 * _N + o, _LANES)]
            dx = txm - txl
            dy = tym - tyl
            dz = tzm - tzl
            b = dx * dx + dy * dy + dz * dz
            new_accs = []
            for smp in range(5):
                pxm = predc[pl.ds((smp * 3 + 0) * _N + o, _LANES)]
                pym = predc[pl.ds((smp * 3 + 1) * _N + o, _LANES)]
                pzm = predc[pl.ds((smp * 3 + 2) * _N + o, _LANES)]
                ex = pxm - psplat[smp * 3 + 0]
                ey = pym - psplat[smp * 3 + 1]
                ez = pzm - psplat[smp * 3 + 2]
                a = ex * ex + ey * ey + ez * ez
                h = 0.5 * (a + b)
                p = a * b
                cnt = accs[smp]
                for ht in _HT2:
                    amt = h - ht
                    cond = amt * jnp.abs(amt) < p
                    cnt = cnt + jnp.where(cond, mv, 0)
                new_accs.append(cnt)
            return (tuple(new_accs), npv + mv)

        return lax.fori_loop(0, _MCHUNKS, m_body, carry1)

    accs, npv = lax.fori_loop(
        0, _ROWS_PER_W, row_body, ((zero16,) * 5, zero16)
    )

    for smp in range(5):
        stage[smp, :] = accs[smp].astype(jnp.float32)
    stage[5, :] = npv.astype(jnp.float32)
    stage[6, :] = jnp.zeros((_LANES,), jnp.float32)
    stage[7, :] = jnp.zeros((_LANES,), jnp.float32)
    pltpu.sync_copy(stage, out_hbm.at[wid])


def _tc_body(pred_l_ref, pred_m_ref, true_l_ref, true_m_ref, mask_ref, out_ref):
    i = pl.program_id(0)

    @pl.when(i == 0)
    def _init():
        out_ref[...] = jnp.zeros_like(out_ref)

    n_sample = pred_l_ref.shape[0]

    txl = true_l_ref[:, 0:1]
    tyl = true_l_ref[:, 1:2]
    tzl = true_l_ref[:, 2:3]
    txm = true_m_ref[0:1, :]
    tym = true_m_ref[1:2, :]
    tzm = true_m_ref[2:3, :]
    dx = txl - txm
    dy = tyl - tym
    dz = tzl - tzm
    dt = jnp.sqrt(dx * dx + dy * dy + dz * dz)

    maskf = mask_ref[...].astype(jnp.float32)
    out_ref[n_sample : n_sample + 1, 0:1] = out_ref[
        n_sample : n_sample + 1, 0:1
    ] + jnp.sum(maskf).reshape(1, 1)

    dtp = dt + (1.0 - maskf) * 1e30
    kc = jnp.int32(0x40FFFFFF)

    for s in range(n_sample):
        pxl = pred_l_ref[s, :, 0:1]
        pyl = pred_l_ref[s, :, 1:2]
        pzl = pred_l_ref[s, :, 2:3]
        pxm = pred_m_ref[s, 0:1, :]
        pym = pred_m_ref[s, 1:2, :]
        pzm = pred_m_ref[s, 2:3, :]
        ex = pxl - pxm
        ey = pyl - pym
        ez = pzl - pzm
        dp = jnp.sqrt(ex * ex + ey * ey + ez * ez)
        err = jnp.abs(dp - dtp)
        u = lax.bitcast_convert_type(err, jnp.int32)
        cnt = jnp.clip((kc - u) >> 23, 0, 4)
        ssum = jnp.sum(cnt.astype(jnp.float32))
        out_ref[s : s + 1, 0:1] = out_ref[s : s + 1, 0:1] + ssum.reshape(1, 1)


def kernel(pred_coordinate, true_coordinate, lddt_mask):
    n_sample = pred_coordinate.shape[0]
    pred_flat = pred_coordinate.reshape(-1)
    true_flat = true_coordinate.reshape(-1)
    pred_t = jnp.transpose(pred_coordinate, (0, 2, 1))
    true_t = true_coordinate.T

    mesh = plsc.VectorSubcoreMesh(core_axis_name="c", subcore_axis_name="s")
    sck = functools.partial(
        pl.kernel,
        out_type=jax.ShapeDtypeStruct((_NW, 8, _LANES), jnp.float32),
        mesh=mesh,
        scratch_types=[
            pltpu.VMEM((15 * _N,), jnp.float32),
            pltpu.VMEM((3 * _N,), jnp.float32),
            pltpu.VMEM((15 * _N,), jnp.float32),
            pltpu.VMEM((3 * _N,), jnp.float32),
            pltpu.VMEM((16, _N), jnp.int32),
            pltpu.VMEM((8, _LANES), jnp.float32),
        ],
        compiler_params=pltpu.CompilerParams(needs_layout_passes=False),
    )(_sc_body)

    sc_parts = sck(pred_flat, true_flat, lddt_mask)

    n_tc_blocks = (_N - _SC_ROWS) // _TC_BLK
    blk0 = _SC_ROWS // _TC_BLK
    tc_out = pl.pallas_call(
        _tc_body,
        grid=(n_tc_blocks,),
        in_specs=[
            pl.BlockSpec((n_sample, _TC_BLK, 3), lambda i: (0, i + blk0, 0)),
            pl.BlockSpec((n_sample, 3, _N), lambda i: (0, 0, 0)),
            pl.BlockSpec((_TC_BLK, 3), lambda i: (i + blk0, 0)),
            pl.BlockSpec((3, _N), lambda i: (0, 0)),
            pl.BlockSpec((_TC_BLK, _N), lambda i: (i + blk0, 0)),
        ],
        out_specs=pl.BlockSpec((8, 128), lambda i: (0, 0)),
        out_shape=jax.ShapeDtypeStruct((8, 128), jnp.float32),
    )(pred_coordinate, pred_t, true_coordinate, true_t, lddt_mask)

    sc_sums = jnp.sum(sc_parts, axis=(0, 2))
    sums = sc_sums[: n_sample + 1] + tc_out[: n_sample + 1, 0]
    return 0.25 * sums[:n_sample] / sums[n_sample]


# speedup vs baseline: 1.0898x; 1.0898x over previous
import functools

import jax
import jax.numpy as jnp
from jax import lax
from jax.experimental import pallas as pl
from jax.experimental.pallas import tpu as pltpu
from jax.experimental.pallas import tpu_sc as plsc

_N = 2048
_LANES = 16
_NC = 2
_NS = 16
_NW = _NC * _NS
_SC_ROWS = 384
_ROWS_PER_W = _SC_ROWS // _NW
_MCHUNKS = _N // _LANES
_HT2 = (0.125, 0.5, 2.0, 8.0)

_TC_BLK = 128


def _sc_body(pred_hbm, true_hbm, mask_hbm, out_hbm,
             rawp, rawt, predc, truec, maskc, stage):
    c = lax.axis_index("c")
    s = lax.axis_index("s")
    wid = s * _NC + c
    row0 = wid * _ROWS_PER_W

    pltpu.sync_copy(pred_hbm, rawp)
    pltpu.sync_copy(true_hbm, rawt)
    win0 = pl.multiple_of((row0 // 8) * 8, 8)
    moff = row0 - win0
    pltpu.sync_copy(mask_hbm.at[pl.ds(win0, 16), :], maskc)

    iota3 = lax.iota(jnp.int32, _LANES) * 3

    def transpose_body(g, carry):
        base = g * (3 * _LANES)
        for d in range(3):
            truec[pl.ds(d * _N + g * _LANES, _LANES)] = plsc.load_gather(
                rawt, [iota3 + (base + d)]
            )
        for smp in range(5):
            for d in range(3):
                predc[pl.ds((smp * 3 + d) * _N + g * _LANES, _LANES)] = (
                    plsc.load_gather(
                        rawp, [iota3 + (smp * 3 * _N + base + d)]
                    )
                )
        return carry

    lax.fori_loop(0, _MCHUNKS, transpose_body, 0)

    zero16 = jnp.zeros((_LANES,), jnp.int32)

    def row_body(r, carry1):
        l3 = (row0 + r) * 3
        lf = jnp.full((_LANES,), l3, jnp.int32)
        txl = plsc.load_gather(rawt, [lf])
        tyl = plsc.load_gather(rawt, [lf + 1])
        tzl = plsc.load_gather(rawt, [lf + 2])
        psplat = []
        for smp in range(5):
            for d in range(3):
                psplat.append(
                    plsc.load_gather(rawp, [lf + (smp * 3 * _N + d)])
                )

        def m_body(mc, carry2):
            accs, npv = carry2
            o = mc * _LANES
            mv = maskc[moff + r, pl.ds(o, _LANES)]
            txm = truec[pl.ds(o, _LANES)]
            tym = truec[pl.ds(_N + o, _LANES)]
            tzm = truec[pl.ds(2 * _N + o, _LANES)]
            dx = txm - txl
            dy = tym - tyl
            dz = tzm - tzl
            b = dx * dx + dy * dy + dz * dz
            new_accs = []
            for smp in range(5):
                pxm = predc[pl.ds((smp * 3 + 0) * _N + o, _LANES)]
                pym = predc[pl.ds((smp * 3 + 1) * _N + o, _LANES)]
                pzm = predc[pl.ds((smp * 3 + 2) * _N + o, _LANES)]
                ex = pxm - psplat[smp * 3 + 0]
                ey = pym - psplat[smp * 3 + 1]
                ez = pzm - psplat[smp * 3 + 2]
                a = ex * ex + ey * ey + ez * ez
                h = 0.5 * (a + b)
                p = a * b
                cnt = accs[smp]
                for ht in _HT2:
                    amt = h - ht
                    cond = amt * jnp.abs(amt) < p
                    cnt = cnt + jnp.where(cond, mv, 0)
                new_accs.append(cnt)
            return (tuple(new_accs), npv + mv)

        return lax.fori_loop(0, _MCHUNKS, m_body, carry1)

    accs, npv = lax.fori_loop(
        0, _ROWS_PER_W, row_body, ((zero16,) * 5, zero16)
    )

    for smp in range(5):
        stage[smp, :] = accs[smp].astype(jnp.float32)
    stage[5, :] = npv.astype(jnp.float32)
    stage[6, :] = jnp.zeros((_LANES,), jnp.float32)
    stage[7, :] = jnp.zeros((_LANES,), jnp.float32)
    pltpu.sync_copy(stage, out_hbm.at[wid])


def _tc_body(pred_l_ref, pred_m_ref, true_l_ref, true_m_ref, mask_ref, out_ref):
    i = pl.program_id(0)

    @pl.when(i == 0)
    def _init():
        out_ref[...] = jnp.zeros_like(out_ref)

    n_sample = pred_l_ref.shape[0]

    txl = true_l_ref[:, 0:1]
    tyl = true_l_ref[:, 1:2]
    tzl = true_l_ref[:, 2:3]
    txm = true_m_ref[0:1, :]
    tym = true_m_ref[1:2, :]
    tzm = true_m_ref[2:3, :]
    dx = txl - txm
    dy = tyl - tym
    dz = tzl - tzm
    dt = jnp.sqrt(dx * dx + dy * dy + dz * dz)

    maskf = mask_ref[...].astype(jnp.float32)
    out_ref[n_sample : n_sample + 1, 0:1] = out_ref[
        n_sample : n_sample + 1, 0:1
    ] + jnp.sum(maskf).reshape(1, 1)

    dtp = dt + (1.0 - maskf) * 1e30
    kc = jnp.int32(0x40FFFFFF)

    for s in range(n_sample):
        pxl = pred_l_ref[s, :, 0:1]
        pyl = pred_l_ref[s, :, 1:2]
        pzl = pred_l_ref[s, :, 2:3]
        pxm = pred_m_ref[s, 0:1, :]
        pym = pred_m_ref[s, 1:2, :]
        pzm = pred_m_ref[s, 2:3, :]
        ex = pxl - pxm
        ey = pyl - pym
        ez = pzl - pzm
        dp = jnp.sqrt(ex * ex + ey * ey + ez * ez)
        err = jnp.abs(dp - dtp)
        u = lax.bitcast_convert_type(err, jnp.int32)
        cnt = jnp.clip((kc - u) >> 23, 0, 4)
        ssum = jnp.sum(cnt.astype(jnp.float32))
        out_ref[s : s + 1, 0:1] = out_ref[s : s + 1, 0:1] + ssum.reshape(1, 1)


def kernel(pred_coordinate, true_coordinate, lddt_mask):
    n_sample = pred_coordinate.shape[0]
    pred_flat = pred_coordinate.reshape(-1)
    true_flat = true_coordinate.reshape(-1)
    pred_t = jnp.transpose(pred_coordinate, (0, 2, 1))
    true_t = true_coordinate.T

    n_tc_blocks = _N // _TC_BLK
    blk0 = 0
    tc_out = pl.pallas_call(
        _tc_body,
        grid=(n_tc_blocks,),
        in_specs=[
            pl.BlockSpec((n_sample, _TC_BLK, 3), lambda i: (0, i + blk0, 0)),
            pl.BlockSpec((n_sample, 3, _N), lambda i: (0, 0, 0)),
            pl.BlockSpec((_TC_BLK, 3), lambda i: (i + blk0, 0)),
            pl.BlockSpec((3, _N), lambda i: (0, 0)),
            pl.BlockSpec((_TC_BLK, _N), lambda i: (i + blk0, 0)),
        ],
        out_specs=pl.BlockSpec((8, 128), lambda i: (0, 0)),
        out_shape=jax.ShapeDtypeStruct((8, 128), jnp.float32),
    )(pred_coordinate, pred_t, true_coordinate, true_t, lddt_mask)

    sums = tc_out[: n_sample + 1, 0]
    return 0.25 * sums[:n_sample] / sums[n_sample]
